# baseline (speedup 1.0000x reference)
import jax
import jax.numpy as jnp
from jax import lax
from jax.experimental import pallas as pl
from jax.experimental.pallas import tpu as pltpu

_TBL_H = 15
_TBL_W = 15
_B = 128
_R = 8


def _stream_kernel(kflat_ref, tbl_ref, x_ref, o_ref, bias_ref, *bufs_and_sems):
    in_bufs = bufs_and_sems[0:_R]
    out_bufs = bufs_and_sems[_R:2 * _R]
    in_sems = bufs_and_sems[2 * _R:3 * _R]
    out_sems = bufs_and_sems[3 * _R:4 * _R]

    kflat = kflat_ref[...]

    def tbl_body(t, acc):
        v = tbl_ref[t // _TBL_W, t % _TBL_W]
        return acc + jnp.where(kflat == t, v, 0.0)

    bias_ref[...] = lax.fori_loop(
        0, _TBL_H * _TBL_W, tbl_body, jnp.zeros(kflat.shape, jnp.float32)
    )

    nsteps = x_ref.shape[0] // _B
    ngroups = nsteps // _R

    def in_copy(i, slot):
        return pltpu.make_async_copy(
            x_ref.at[pl.ds(i * _B, _B)], in_bufs[slot], in_sems[slot])

    def out_copy(i, slot):
        return pltpu.make_async_copy(
            out_bufs[slot], o_ref.at[pl.ds(i * _B, _B)], out_sems[slot])

    for r in range(_R):
        in_copy(r, r).start()

    def group(g, carry):
        base = g * _R
        for r in range(_R):
            i = base + r
            in_copy(i, r).wait()

            @pl.when(g > 0)
            def _wait_out_slot():
                out_copy(i - _R, r).wait()

            out_bufs[r][...] = in_bufs[r][...] + bias_ref[...]
            out_copy(i, r).start()

            @pl.when(g < ngroups - 1)
            def _prefetch():
                in_copy(i + _R, r).start()

        return carry

    lax.fori_loop(0, ngroups, group, 0)

    for r in range(_R):
        out_copy(nsteps - _R + r, r).wait()


def kernel(x, pos_embeddings, relative_indices):
    nb, nh, h, w = x.shape
    n = nb * nh
    hw = h * w
    x2 = x.reshape(n, hw)
    kflat = (relative_indices[:, :, 0] * _TBL_W + relative_indices[:, :, 1]).reshape(1, hw)

    buf = pltpu.VMEM((_B, hw), jnp.float32)
    out = pl.pallas_call(
        _stream_kernel,
        in_specs=[
            pl.BlockSpec(memory_space=pltpu.VMEM),
            pl.BlockSpec(memory_space=pltpu.SMEM),
            pl.BlockSpec(memory_space=pl.ANY),
        ],
        out_specs=pl.BlockSpec(memory_space=pl.ANY),
        out_shape=jax.ShapeDtypeStruct((n, hw), jnp.float32),
        scratch_shapes=(
            [pltpu.VMEM((1, hw), jnp.float32)]
            + [buf] * (2 * _R)
            + [pltpu.SemaphoreType.DMA] * (2 * _R)
        ),
    )(kflat, pos_embeddings, x2)
    return out.reshape(x.shape)


# speedup vs baseline: 1.2535x; 1.2535x over previous
import jax
import jax.numpy as jnp
from jax import lax
from jax.experimental import pallas as pl
from jax.experimental.pallas import tpu as pltpu

_TBL_H = 15
_TBL_W = 15
_B = 4
_R = 8
_NPRI = 2


def _stream_kernel(i0_ref, i1_ref, tbl_ref, x_ref, o_ref, bias_ref, *bufs_and_sems):
    in_bufs = bufs_and_sems[0:_R]
    out_bufs = bufs_and_sems[_R:2 * _R]
    in_sems = bufs_and_sems[2 * _R:3 * _R]
    out_sems = bufs_and_sems[3 * _R:4 * _R]

    kflat = i0_ref[...] * _TBL_W + i1_ref[...]

    def tbl_body(t, acc):
        v = tbl_ref[t // _TBL_W, t % _TBL_W]
        return acc + jnp.where(kflat == t, v, 0.0)

    bias_ref[...] = lax.fori_loop(
        0, _TBL_H * _TBL_W, tbl_body, jnp.zeros(kflat.shape, jnp.float32)
    )

    nsteps = x_ref.shape[0] // _B
    ngroups = nsteps // _R

    def in_copy(i, slot):
        return pltpu.make_async_copy(
            x_ref.at[pl.ds(i * _B, _B)], in_bufs[slot], in_sems[slot])

    def out_copy(i, slot):
        return pltpu.make_async_copy(
            out_bufs[slot], o_ref.at[pl.ds(i * _B, _B)], out_sems[slot])

    for r in range(_R):
        in_copy(r, r).start(priority=r % _NPRI)

    def group(g, carry):
        base = g * _R
        for r in range(_R):
            i = base + r
            in_copy(i, r).wait()

            @pl.when(g > 0)
            def _wait_out_slot():
                out_copy(i - _R, r).wait()

            out_bufs[r][...] = in_bufs[r][...] + bias_ref[...][None, None, :, :]
            out_copy(i, r).start(priority=r % _NPRI)

            @pl.when(g < ngroups - 1)
            def _prefetch():
                in_copy(i + _R, r).start(priority=r % _NPRI)

        return carry

    lax.fori_loop(0, ngroups, group, 0)

    for r in range(_R):
        out_copy(nsteps - _R + r, r).wait()


def kernel(x, pos_embeddings, relative_indices):
    nb, nh, h, w = x.shape
    i0 = relative_indices[:, :, 0]
    i1 = relative_indices[:, :, 1]

    buf = pltpu.VMEM((_B, nh, h, w), jnp.float32)
    out = pl.pallas_call(
        _stream_kernel,
        in_specs=[
            pl.BlockSpec(memory_space=pltpu.VMEM),
            pl.BlockSpec(memory_space=pltpu.VMEM),
            pl.BlockSpec(memory_space=pltpu.SMEM),
            pl.BlockSpec(memory_space=pl.ANY),
        ],
        out_specs=pl.BlockSpec(memory_space=pl.ANY),
        out_shape=jax.ShapeDtypeStruct(x.shape, jnp.float32),
        scratch_shapes=(
            [pltpu.VMEM((h, w), jnp.float32)]
            + [buf] * (2 * _R)
            + [pltpu.SemaphoreType.DMA] * (2 * _R)
        ),
    )(i0, i1, pos_embeddings, x)
    return out
